# baseline (speedup 1.0000x reference)
import jax
import jax.numpy as jnp
from jax import lax
from jax.experimental import pallas as pl
from jax.experimental.pallas import tpu as pltpu

OUT_D = 1000
B, L = 4096, 20
B0 = 256
NBLK = B // B0


def _body(x_ref, o_ref):
    xb = x_ref[...]
    iota = lax.broadcasted_iota(jnp.int32, (B0, L, OUT_D), 2)
    o_ref[...] = (iota == xb[:, :, None]).astype(jnp.float32)


def kernel(x):
    return pl.pallas_call(
        _body,
        grid=(NBLK,),
        in_specs=[pl.BlockSpec((B0, L), lambda i: (i, 0))],
        out_specs=pl.BlockSpec((B0, L, OUT_D), lambda i: (i, 0, 0)),
        out_shape=jax.ShapeDtypeStruct((B, L, OUT_D), jnp.float32),
        compiler_params=pltpu.CompilerParams(
            dimension_semantics=("parallel",),
        ),
    )(x)


# speedup vs baseline: 1.0102x; 1.0102x over previous
import jax
import jax.numpy as jnp
from jax import lax
from jax.experimental import pallas as pl
from jax.experimental.pallas import tpu as pltpu

OUT_D = 1000
B, L = 4096, 20
B0 = 64
NBLK = B // B0
NBUF = 6


def _body(x_ref, o_ref, scratch, sems):
    def step(i, carry):
        slot = lax.rem(i, NBUF)

        @pl.when(i >= NBUF)
        def _wait_prev():
            pltpu.make_async_copy(
                scratch.at[slot],
                o_ref.at[pl.ds((i - NBUF) * B0, B0)],
                sems.at[slot],
            ).wait()

        xb = x_ref[pl.ds(i * B0, B0), :]
        iota = lax.broadcasted_iota(jnp.int32, (B0, L, OUT_D), 2)
        scratch[slot] = (iota == xb[:, :, None]).astype(jnp.float32)
        pltpu.make_async_copy(
            scratch.at[slot],
            o_ref.at[pl.ds(i * B0, B0)],
            sems.at[slot],
        ).start()
        return carry

    lax.fori_loop(0, NBLK, step, 0)

    def drain(i, carry):
        slot = lax.rem(i, NBUF)
        pltpu.make_async_copy(
            scratch.at[slot],
            o_ref.at[pl.ds(i * B0, B0)],
            sems.at[slot],
        ).wait()
        return carry

    lax.fori_loop(NBLK - NBUF, NBLK, drain, 0)


def kernel(x):
    return pl.pallas_call(
        _body,
        in_specs=[pl.BlockSpec(memory_space=pltpu.MemorySpace.VMEM)],
        out_specs=pl.BlockSpec(memory_space=pltpu.MemorySpace.HBM),
        out_shape=jax.ShapeDtypeStruct((B, L, OUT_D), jnp.float32),
        scratch_shapes=[
            pltpu.VMEM((NBUF, B0, L, OUT_D), jnp.float32),
            pltpu.SemaphoreType.DMA((NBUF,)),
        ],
    )(x)
